# Initial kernel scaffold; baseline (speedup 1.0000x reference)
import functools

import jax
import jax.numpy as jnp
from jax import lax
from jax.experimental import pallas as pl
from jax.experimental.pallas import tpu as pltpu
from jax.experimental.pallas import tpu_sc as plsc

N0, N1, N2 = 50000, 10000, 4096
E1, E2 = 320000, 131072
D = 128
DP = 144
NC, NS = 2, 16


def _make_sc_agg(E, N, C, interpret=False):
    EW = E // (NC * NS)
    n_chunks = EW // C
    assert n_chunks * C == EW
    RPT = N // NS
    assert RPT * NS == N
    mesh = plsc.VectorSubcoreMesh(core_axis_name="c", subcore_axis_name="s")

    @functools.partial(
        pl.kernel,
        out_type=jax.ShapeDtypeStruct((NC, N, DP), jnp.float32),
        mesh=mesh,
        scratch_types=[
            pltpu.VMEM((2, C), jnp.int32),
            pltpu.VMEM((C, DP), jnp.float32),
            pltpu.VMEM_SHARED((N, DP), jnp.float32),
            pltpu.SemaphoreType.DMA,
        ],
        interpret=interpret,
    )
    def agg_kernel(table, src, dst, zeros, out, idx_v, rows_v, acc_sh, sem):
        cid = lax.axis_index("c")
        sid = lax.axis_index("s")
        base = (cid * NS + sid) * EW
        pltpu.sync_copy(zeros.at[pl.ds(sid * RPT, RPT)],
                        acc_sh.at[pl.ds(sid * RPT, RPT)])
        plsc.subcore_barrier()

        def body(i, carry):
            off = base + i * C
            pltpu.sync_copy(src.at[pl.ds(off, C)], idx_v.at[0])
            pltpu.sync_copy(dst.at[pl.ds(off, C)], idx_v.at[1])
            pltpu.async_copy(table.at[idx_v.at[0]], rows_v, sem).wait()
            pltpu.sync_copy(rows_v, acc_sh.at[idx_v.at[1]], add=True)
            return carry

        lax.fori_loop(0, n_chunks, body, 0)
        plsc.subcore_barrier()
        pltpu.sync_copy(acc_sh.at[pl.ds(sid * RPT, RPT)],
                        out.at[cid, pl.ds(sid * RPT, RPT)])

    return agg_kernel


def _dense(parts, xdst, wlT, wrT, b, relu, pad_out, BR, interpret=False):
    N = xdst.shape[0]
    assert N % BR == 0
    DO = DP if pad_out else D

    def body(p_ref, xd_ref, wl_ref, wr_ref, b_ref, o_ref):
        agg = p_ref[0] + p_ref[1]
        cnt = jnp.maximum(agg[:, D:D + 1], 1.0)
        mean = agg[:, :D] / cnt
        h = jnp.dot(mean, wl_ref[...], preferred_element_type=jnp.float32)
        h = h + jnp.dot(xd_ref[...], wr_ref[...],
                        preferred_element_type=jnp.float32)
        h = h + b_ref[...]
        if relu:
            h = jnp.maximum(h, 0.0)
        if pad_out:
            col = lax.broadcasted_iota(jnp.int32, (BR, DP - D), 1) == 0
            h = jnp.concatenate([h, col.astype(jnp.float32)], axis=1)
        o_ref[...] = h

    return pl.pallas_call(
        body,
        grid=(N // BR,),
        in_specs=[
            pl.BlockSpec((NC, BR, DP), lambda i: (0, i, 0)),
            pl.BlockSpec((BR, D), lambda i: (i, 0)),
            pl.BlockSpec((D, D), lambda i: (0, 0)),
            pl.BlockSpec((D, D), lambda i: (0, 0)),
            pl.BlockSpec((1, D), lambda i: (0, 0)),
        ],
        out_specs=pl.BlockSpec((BR, DO), lambda i: (i, 0)),
        out_shape=jax.ShapeDtypeStruct((N, DO), jnp.float32),
        interpret=interpret,
    )(parts, xdst, wlT, wrT, b)


def kernel(x, edge_index1, edge_index2, W_l1, b_l1, W_r1, W_l2, b_l2, W_r2):
    src1 = edge_index1[0].astype(jnp.int32)
    dst1 = edge_index1[1].astype(jnp.int32)
    src2 = edge_index2[0].astype(jnp.int32)
    dst2 = edge_index2[1].astype(jnp.int32)

    onescol = (jnp.arange(DP - D)[None, :] == 0).astype(jnp.float32)
    xe = jnp.concatenate([x, jnp.broadcast_to(onescol, (N0, DP - D))], axis=1)
    z1 = jnp.zeros((N1, DP), jnp.float32)
    z2 = jnp.zeros((N2, DP), jnp.float32)

    parts1 = _make_sc_agg(E1, N1, 80)(xe, src1, dst1, z1)
    he = _dense(parts1, x[:N1], W_l1.T, W_r1.T, b_l1[None, :],
                relu=True, pad_out=True, BR=1000)
    parts2 = _make_sc_agg(E2, N2, 128)(he, src2, dst2, z2)
    h2 = _dense(parts2, he[:N2, :D], W_l2.T, W_r2.T, b_l2[None, :],
                relu=False, pad_out=False, BR=1024)
    out = he[:, :D]
    return (h2, h2, out)


# speedup vs baseline: 5.0709x; 5.0709x over previous
import functools

import jax
import jax.numpy as jnp
from jax import lax
from jax.experimental import pallas as pl
from jax.experimental.pallas import tpu as pltpu
from jax.experimental.pallas import tpu_sc as plsc

N0, N1, N2 = 50000, 10000, 4096
E1, E2 = 320000, 131072
D = 128
DP = 144
NC, NS = 2, 16


def _make_sc_agg(E, NP, C, interpret=False):
    EW = E // (NC * NS)
    n_chunks = EW // C
    assert n_chunks * C == EW
    RPT = NP // NS
    assert RPT * NS == NP and RPT % 8 == 0
    mesh = plsc.VectorSubcoreMesh(core_axis_name="c", subcore_axis_name="s",
                                  num_cores=NC, num_subcores=NS)

    @functools.partial(
        pl.kernel,
        out_type=jax.ShapeDtypeStruct((NC, NP, DP), jnp.float32),
        mesh=mesh,
        scratch_types=[
            pltpu.VMEM((2, C), jnp.int32),
            pltpu.VMEM((C, DP), jnp.float32),
            pltpu.VMEM_SHARED((NP, DP), jnp.float32),
            pltpu.SemaphoreType.DMA,
        ],
        compiler_params=pltpu.CompilerParams(use_tc_tiling_on_sc=False),
        interpret=interpret,
    )
    def agg_kernel(table, src, dst, zeros, out, idx_v, rows_v, acc_sh, sem):
        cid = lax.axis_index("c")
        sid = lax.axis_index("s")
        base = (cid * NS + sid) * EW
        pltpu.sync_copy(zeros.at[pl.ds(sid * RPT, RPT)],
                        acc_sh.at[pl.ds(sid * RPT, RPT)])
        plsc.subcore_barrier()

        def body(i, carry):
            off = base + i * C
            pltpu.sync_copy(src.at[pl.ds(off, C)], idx_v.at[0])
            pltpu.sync_copy(dst.at[pl.ds(off, C)], idx_v.at[1])
            pltpu.async_copy(table.at[idx_v.at[0]], rows_v, sem).wait()
            pltpu.sync_copy(rows_v, acc_sh.at[idx_v.at[1]], add=True)
            return carry

        lax.fori_loop(0, n_chunks, body, 0)
        plsc.subcore_barrier()
        pltpu.sync_copy(acc_sh.at[pl.ds(sid * RPT, RPT)],
                        out.at[cid, pl.ds(sid * RPT, RPT)])

    return agg_kernel


def _dense(parts, xdst, wlT, wrT, b, relu, pad_out, BR, interpret=False):
    N = xdst.shape[0]
    assert N % BR == 0
    DO = DP if pad_out else D

    def body(p_ref, xd_ref, wl_ref, wr_ref, b_ref, o_ref):
        agg = p_ref[0] + p_ref[1]
        cnt = jnp.maximum(agg[:, D:D + 1], 1.0)
        mean = agg[:, :D] / cnt
        h = jnp.dot(mean, wl_ref[...], preferred_element_type=jnp.float32)
        h = h + jnp.dot(xd_ref[...], wr_ref[...],
                        preferred_element_type=jnp.float32)
        h = h + b_ref[...]
        if relu:
            h = jnp.maximum(h, 0.0)
        if pad_out:
            col = lax.broadcasted_iota(jnp.int32, (BR, DP - D), 1) == 0
            h = jnp.concatenate([h, col.astype(jnp.float32)], axis=1)
        o_ref[...] = h

    return pl.pallas_call(
        body,
        grid=(N // BR,),
        in_specs=[
            pl.BlockSpec((NC, BR, DP), lambda i: (0, i, 0)),
            pl.BlockSpec((BR, D), lambda i: (i, 0)),
            pl.BlockSpec((D, D), lambda i: (0, 0)),
            pl.BlockSpec((D, D), lambda i: (0, 0)),
            pl.BlockSpec((1, D), lambda i: (0, 0)),
        ],
        out_specs=pl.BlockSpec((BR, DO), lambda i: (i, 0)),
        out_shape=jax.ShapeDtypeStruct((N, DO), jnp.float32),
        interpret=interpret,
    )(parts, xdst, wlT, wrT, b)


def kernel(x, edge_index1, edge_index2, W_l1, b_l1, W_r1, W_l2, b_l2, W_r2):
    src1 = edge_index1[0].astype(jnp.int32)
    dst1 = edge_index1[1].astype(jnp.int32)
    src2 = edge_index2[0].astype(jnp.int32)
    dst2 = edge_index2[1].astype(jnp.int32)

    onescol = (jnp.arange(DP - D)[None, :] == 0).astype(jnp.float32)
    xe = jnp.concatenate([x, jnp.broadcast_to(onescol, (N0, DP - D))], axis=1)
    N1P = 10112
    z1 = jnp.zeros((N1P, DP), jnp.float32)
    z2 = jnp.zeros((N2, DP), jnp.float32)

    parts1 = _make_sc_agg(E1, N1P, 80)(xe, src1, dst1, z1)
    he = _dense(parts1, x[:N1], W_l1.T, W_r1.T, b_l1[None, :],
                relu=True, pad_out=True, BR=1000)
    parts2 = _make_sc_agg(E2, N2, 128)(he, src2, dst2, z2)
    h2 = _dense(parts2, he[:N2, :D], W_l2.T, W_r2.T, b_l2[None, :],
                relu=False, pad_out=False, BR=1024)
    out = he[:, :D]
    return (h2, h2, out)
